# baseline (speedup 1.0000x reference)
import math

import jax
import jax.numpy as jnp
from jax import lax
from jax.experimental import pallas as pl
from jax.experimental.pallas import tpu as pltpu
from jax.experimental.pallas import tpu_sc as plsc

D_MODEL = 64
SCALE = math.sqrt(D_MODEL)
OUT_W = 128

NC = 2
NS = 16
NW = NC * NS
L = 16

BLK = 128
R = 8


def _encoder_body(sj_hbm, slot_hbm, f_hbm, fb_hbm, fs_hbm, pb_hbm, orow_hbm,
                  wt_hbm, out_hbm,
                  sj_v, slot_v, f_v, fb_v, fs_v, pb_v, orow_v, out_v,
                  *slabs_sems):
    slabs = slabs_sems[:R]
    sems = slabs_sems[R:2 * R]
    osem = slabs_sems[2 * R]
    n_per_w = sj_hbm.shape[1]
    wid = lax.axis_index("s") * NC + lax.axis_index("c")

    pltpu.sync_copy(sj_hbm.at[wid], sj_v)
    pltpu.sync_copy(slot_hbm.at[wid], slot_v)
    pltpu.sync_copy(f_hbm.at[wid], f_v)
    pltpu.sync_copy(fb_hbm.at[wid], fb_v)
    pltpu.sync_copy(fs_hbm.at[wid], fs_v)
    pltpu.sync_copy(pb_hbm.at[wid], pb_v)
    pltpu.sync_copy(orow_hbm.at[wid], orow_v)

    iota = lax.iota(jnp.int32, L)

    def fire(b, r):
        blk0 = pl.multiple_of(b << 7, BLK)
        pltpu.async_copy(
            wt_hbm.at[:, pl.ds(blk0, BLK)], slabs[r], sems[r]
        )

    pb = pb_v[pl.ds(0, L)]
    for n in range(R - 1):
        @pl.when(pb[n] >= 0)
        def _(n=n):
            fire(pb[n], n % R)

    def loop_body(g, carry):
        base = g * L
        sjv = sj_v[pl.ds(base, L)]
        slv = slot_v[pl.ds(base, L)]
        fv = f_v[pl.ds(base, L)]
        fbv = fb_v[pl.ds(base, L)]
        fsv = fs_v[pl.ds(base, L)]
        for k in range(L):
            i = base + k
            s = slv[k]

            @pl.when(fbv[k] >= 0)
            def _():
                for rr in range(R):
                    @pl.when(fsv[k] == rr)
                    def _(rr=rr):
                        fire(fbv[k], rr)

            @pl.when(fv[k] == 1)
            def _():
                for rr in range(R):
                    @pl.when(s == rr)
                    def _(rr=rr):
                        pltpu.make_async_copy(
                            wt_hbm.at[:, pl.ds(0, BLK)], slabs[rr], sems[rr]
                        ).wait()

            j_vec = jnp.full((L,), sjv[k], jnp.int32)
            for rr in range(R):
                @pl.when(s == rr)
                def _(rr=rr):
                    for c0 in range(0, D_MODEL, L):
                        vals = plsc.load_gather(slabs[rr], [c0 + iota, j_vec])
                        out_v[i, pl.ds(c0, L)] = vals * SCALE

        return carry

    lax.fori_loop(0, n_per_w // L, loop_body, 0)

    for h in range(2):
        pltpu.async_copy(
            out_v.at[pl.ds(h * BLK, BLK)], out_hbm.at[orow_v.at[h]], osem
        )
    pltpu.make_async_copy(
        out_v, out_hbm.at[pl.ds(0, 2 * BLK)], osem
    ).wait()


def kernel(src, weight):
    seq_len, batch = src.shape
    n_tokens, d_model = weight.shape
    b_total = seq_len * batch
    assert d_model == D_MODEL
    n_per_w = b_total // NW
    assert b_total % NW == 0 and n_per_w == 2 * BLK

    flat = src.reshape(-1).astype(jnp.int32)
    st, order = lax.sort(
        [flat, jnp.arange(b_total, dtype=jnp.int32)], num_keys=1
    )
    sb = (st >> 7).reshape(NW, n_per_w)
    sj = (st & (BLK - 1)).reshape(NW, n_per_w)

    pos = jnp.arange(n_per_w, dtype=jnp.int32)[None, :]
    nf = jnp.concatenate(
        [jnp.ones((NW, 1), bool), sb[:, 1:] != sb[:, :-1]], axis=1
    )
    lc = lax.cummax(jnp.where(nf, pos, 0), axis=1)
    f = nf | ((pos - lc) % R == 0)
    fidx = jnp.cumsum(f.astype(jnp.int32), axis=1) - 1
    cnt = jnp.sum(f.astype(jnp.int32), axis=1)
    rows = jnp.broadcast_to(
        jnp.arange(NW, dtype=jnp.int32)[:, None], (NW, n_per_w)
    )
    bof = (
        jnp.zeros((NW, n_per_w), jnp.int32)
        .at[rows, jnp.where(f, fidx, n_per_w)]
        .set(sb, mode="drop")
    )
    tgt = fidx + (R - 1)
    fire_b = jnp.where(
        f & (tgt < cnt[:, None]),
        jnp.take_along_axis(bof, jnp.clip(tgt, 0, n_per_w - 1), axis=1),
        -1,
    ).astype(jnp.int32)
    fire_s = ((fidx - 1) % R).astype(jnp.int32)
    slot = (fidx % R).astype(jnp.int32)
    prime_n = jnp.arange(L, dtype=jnp.int32)[None, :]
    prime_b = jnp.where(
        prime_n < jnp.minimum(R - 1, cnt)[:, None], bof[:, :L], -1
    ).astype(jnp.int32)
    orow = order.reshape(NW, 2, BLK)

    wt = weight.T

    gather = pl.kernel(
        _encoder_body,
        out_type=jax.ShapeDtypeStruct((b_total, OUT_W), jnp.float32),
        mesh=plsc.VectorSubcoreMesh(
            core_axis_name="c", subcore_axis_name="s",
            num_cores=NC, num_subcores=NS,
        ),
        scratch_types=(
            [
                pltpu.VMEM((n_per_w,), jnp.int32),
                pltpu.VMEM((n_per_w,), jnp.int32),
                pltpu.VMEM((n_per_w,), jnp.int32),
                pltpu.VMEM((n_per_w,), jnp.int32),
                pltpu.VMEM((n_per_w,), jnp.int32),
                pltpu.VMEM((L,), jnp.int32),
                pltpu.VMEM((2, BLK), jnp.int32),
                pltpu.VMEM((n_per_w, OUT_W), jnp.float32),
            ]
            + [pltpu.VMEM((D_MODEL, BLK), jnp.float32) for _ in range(R)]
            + [pltpu.SemaphoreType.DMA for _ in range(R + 1)]
        ),
        compiler_params=pltpu.CompilerParams(
            use_tc_tiling_on_sc=True, needs_layout_passes=False
        ),
    )
    out = gather(
        sj, slot, f.astype(jnp.int32), fire_b, fire_s, prime_b, orow, wt
    )
    return out[:, :d_model].reshape(seq_len, batch, d_model)


# speedup vs baseline: 1.4218x; 1.4218x over previous
import math

import jax
import jax.numpy as jnp
from jax import lax
from jax.experimental import pallas as pl
from jax.experimental.pallas import tpu as pltpu
from jax.experimental.pallas import tpu_sc as plsc

D_MODEL = 64
SCALE = math.sqrt(D_MODEL)
OUT_W = 128

NC = 2
NS = 16
NW = NC * NS
L = 16

BLK = 128
R = 8


def _encoder_body(idx_hbm, wt_hbm, out_hbm, idx_v, out_v, *slabs_sems):
    slabs = slabs_sems[:R]
    sems = slabs_sems[R:2 * R]
    osem = slabs_sems[2 * R]
    n_per_w = idx_hbm.shape[1]
    wid = lax.axis_index("s") * NC + lax.axis_index("c")

    pltpu.sync_copy(idx_hbm.at[wid], idx_v.at[pl.ds(0, n_per_w)])

    iota = lax.iota(jnp.int32, L)

    def fire(t, r):
        blk0 = pl.multiple_of((t >> 7) << 7, BLK)
        pltpu.async_copy(
            wt_hbm.at[:, pl.ds(blk0, BLK)], slabs[r], sems[r]
        )

    def extract(t, i, r):
        j_vec = jnp.full((L,), t & (BLK - 1), jnp.int32)
        for c0 in range(0, D_MODEL, L):
            vals = plsc.load_gather(slabs[r], [c0 + iota, j_vec])
            out_v[i, pl.ds(c0, L)] = vals * SCALE

    t_prime = idx_v[pl.ds(0, L)]
    for r in range(R):
        fire(t_prime[r], r)

    def loop_body(g, carry):
        base = g * L
        t_cur = idx_v[pl.ds(base, L)]
        t_nxt = idx_v[pl.ds(base + L, L)]
        for k in range(L):
            i = base + k
            r = k % R
            pltpu.make_async_copy(
                wt_hbm.at[:, pl.ds(0, BLK)], slabs[r], sems[r]
            ).wait()
            extract(t_cur[k], i, r)
            t_ahead = t_cur[k + R] if k + R < L else t_nxt[k + R - L]

            @pl.when(i + R < n_per_w)
            def _():
                fire(t_ahead, r)

        return carry

    lax.fori_loop(0, n_per_w // L, loop_body, 0)

    pltpu.async_copy(
        out_v, out_hbm.at[pl.ds(wid * n_per_w, n_per_w)], osem
    ).wait()


def kernel(src, weight):
    seq_len, batch = src.shape
    n_tokens, d_model = weight.shape
    b_total = seq_len * batch
    assert d_model == D_MODEL
    n_per_w = b_total // NW
    assert b_total % NW == 0 and n_per_w % R == 0

    idx = src.reshape(NW, n_per_w).astype(jnp.int32)
    wt = weight.T

    gather = pl.kernel(
        _encoder_body,
        out_type=jax.ShapeDtypeStruct((b_total, OUT_W), jnp.float32),
        mesh=plsc.VectorSubcoreMesh(
            core_axis_name="c", subcore_axis_name="s",
            num_cores=NC, num_subcores=NS,
        ),
        scratch_types=(
            [
                pltpu.VMEM((n_per_w + L,), jnp.int32),
                pltpu.VMEM((n_per_w, OUT_W), jnp.float32),
            ]
            + [pltpu.VMEM((D_MODEL, BLK), jnp.float32) for _ in range(R)]
            + [pltpu.SemaphoreType.DMA for _ in range(R + 1)]
        ),
        compiler_params=pltpu.CompilerParams(
            use_tc_tiling_on_sc=True, needs_layout_passes=False
        ),
    )
    out = gather(idx, wt)
    return out[:, :d_model].reshape(seq_len, batch, d_model)
